# baseline (speedup 1.0000x reference)
import functools

import jax
import jax.numpy as jnp
from jax import lax
from jax.experimental import pallas as pl
from jax.experimental.pallas import tpu as pltpu
from jax.experimental.pallas import tpu_sc as plsc

MAX_CTR_F = 100000.0
SIZE_P = 64
BATCH = 4096
MAX_CLICKED = 200

_NC, _NS, _LANES = 2, 16, 16
_NW = _NC * _NS
_BBLK = BATCH // _NW
_ST = MAX_CLICKED // 8
_PHASES = ((0, 13), (13, 12))


def _sc_body(t0, tp, ctr4_hbm, table_hbm, out_hbm, cbuf, idxbuf,
             ebuf0, obuf0, ebuf1, obuf1, sem0, sem1):
    w = lax.axis_index("s") * _NC + lax.axis_index("c")
    nq = 4 * tp
    obase = w * (nq * _BBLK)

    pltpu.sync_copy(ctr4_hbm.at[pl.ds(t0, tp), w], cbuf)

    def quant(t, carry):
        for i in range(8):
            for k in range(_BBLK // _LANES):
                sl = pl.ds(k * _LANES, _LANES)
                idxbuf[t, i, sl] = (cbuf[t, i, sl] * MAX_CTR_F).astype(jnp.int32)
        return carry

    lax.fori_loop(0, tp, quant, 0)

    ebufs = (ebuf0, ebuf1)
    obufs = (obuf0, obuf1)
    sems = (sem0, sem1)

    def fire(c, b):
        t = c // 4
        i = 2 * (c % 4)
        pltpu.async_copy(table_hbm.at[idxbuf.at[t, i]], ebufs[b], sems[b])
        pltpu.async_copy(table_hbm.at[idxbuf.at[t, i + 1]], obufs[b], sems[b])

    def drain(b):
        dummy = table_hbm.at[pl.ds(0, _BBLK)]
        pltpu.make_async_copy(dummy, ebufs[b], sems[b]).wait()
        pltpu.make_async_copy(dummy, obufs[b], sems[b]).wait()

    fire(0, 0)
    fire(1, 1)

    def step(c2, carry):
        for b in range(2):
            c = 2 * c2 + b
            drain(b)
            rows = pl.ds(obase + c * _BBLK, _BBLK)
            pltpu.sync_copy(ebufs[b], out_hbm.at[rows, pl.ds(0, SIZE_P)])
            pltpu.sync_copy(obufs[b], out_hbm.at[rows, pl.ds(SIZE_P, SIZE_P)])

            @pl.when(c + 2 < nq)
            def _():
                fire(c + 2, b)

        return carry

    lax.fori_loop(0, nq // 2, step, 0)


def _sc_gather(ctr4, table, phase):
    t0, tp = _PHASES[phase]
    mesh = plsc.VectorSubcoreMesh(core_axis_name="c", subcore_axis_name="s")
    k = pl.kernel(
        functools.partial(_sc_body, t0, tp),
        jax.ShapeDtypeStruct((_NW * 4 * tp * _BBLK, 2 * SIZE_P), jnp.float32),
        mesh=mesh,
        scratch_types=[
            pltpu.VMEM((tp, 8, _BBLK), jnp.float32),
            pltpu.VMEM((tp, 8, _BBLK), jnp.int32),
            pltpu.VMEM((_BBLK, SIZE_P), jnp.float32),
            pltpu.VMEM((_BBLK, SIZE_P), jnp.float32),
            pltpu.VMEM((_BBLK, SIZE_P), jnp.float32),
            pltpu.VMEM((_BBLK, SIZE_P), jnp.float32),
            pltpu.SemaphoreType.DMA,
            pltpu.SemaphoreType.DMA,
        ],
        compiler_params=pltpu.CompilerParams(use_tc_tiling_on_sc=False),
    )
    return k(ctr4, table)


def _tc_transpose(x128, phase, prev=None):
    t0, tp = _PHASES[phase]

    def body(x_ref, o_ref):
        for q in range(4):
            blk = x_ref[pl.ds(q * _BBLK, _BBLK), :].T
            o_ref[pl.ds(2 * q, 2), :, :] = blk.reshape(2, SIZE_P, _BBLK)

    def body_aliased(x_ref, prev_ref, o_ref):
        del prev_ref
        body(x_ref, o_ref)

    out_shape = jax.ShapeDtypeStruct((MAX_CLICKED, SIZE_P, BATCH), jnp.float32)
    x_spec = pl.BlockSpec((4 * _BBLK, 2 * SIZE_P),
                          lambda i, j, n=tp: (i * n + j, 0))
    o_spec = pl.BlockSpec((8, SIZE_P, _BBLK),
                          lambda i, j, s0=t0: (s0 + j, 0, i))
    if prev is None:
        return pl.pallas_call(
            body, grid=(_NW, tp), in_specs=[x_spec], out_specs=o_spec,
            out_shape=out_shape,
        )(x128)
    return pl.pallas_call(
        body_aliased, grid=(_NW, tp),
        in_specs=[x_spec, pl.BlockSpec(memory_space=pltpu.MemorySpace.HBM)],
        out_specs=o_spec, out_shape=out_shape,
        input_output_aliases={1: 0},
    )(x128, prev)


@jax.jit
def kernel(ctr, embedding_table):
    ctr4 = ctr.reshape(32, 128, 25, 8).transpose(2, 0, 3, 1)
    inters = [_sc_gather(ctr4, embedding_table, p) for p in range(len(_PHASES))]
    phys = _tc_transpose(inters[0], 0)
    for p in range(1, len(_PHASES)):
        phys = _tc_transpose(inters[p], p, prev=phys)
    return phys.transpose(2, 0, 1)


# speedup vs baseline: 1.8341x; 1.8341x over previous
import functools

import jax
import jax.numpy as jnp
from jax import lax
from jax.experimental import pallas as pl
from jax.experimental.pallas import tpu as pltpu
from jax.experimental.pallas import tpu_sc as plsc

MAX_CTR_F = 100000.0
SIZE_P = 64
BATCH = 4096
MAX_CLICKED = 200

_NC, _NS, _LANES = 2, 16, 16
_NW = _NC * _NS
_BBLK = BATCH // _NW
_ST = MAX_CLICKED // 8
_PHASES = ((0, 5), (5, 5), (10, 5), (15, 5), (20, 5))


def _sc_body(t0, tp, ctr4_hbm, table_hbm, out_hbm, cbuf, idxbuf,
             ebuf0, obuf0, ebuf1, obuf1, sem0, sem1):
    w = lax.axis_index("s") * _NC + lax.axis_index("c")
    nq = 4 * tp
    obase = w * (nq * _BBLK)

    pltpu.sync_copy(ctr4_hbm.at[pl.ds(t0, tp), w], cbuf)

    def quant(t, carry):
        for i in range(8):
            for k in range(_BBLK // _LANES):
                sl = pl.ds(k * _LANES, _LANES)
                idxbuf[t, i, sl] = (cbuf[t, i, sl] * MAX_CTR_F).astype(jnp.int32)
        return carry

    lax.fori_loop(0, tp, quant, 0)

    ebufs = (ebuf0, ebuf1)
    obufs = (obuf0, obuf1)
    sems = (sem0, sem1)

    def fire(c, b):
        t = c // 4
        i = 2 * (c % 4)
        pltpu.async_copy(table_hbm.at[idxbuf.at[t, i]], ebufs[b], sems[b])
        pltpu.async_copy(table_hbm.at[idxbuf.at[t, i + 1]], obufs[b], sems[b])

    def drain(b):
        dummy = table_hbm.at[pl.ds(0, _BBLK)]
        pltpu.make_async_copy(dummy, ebufs[b], sems[b]).wait()
        pltpu.make_async_copy(dummy, obufs[b], sems[b]).wait()

    fire(0, 0)
    fire(1, 1)

    def step(c2, carry):
        for b in range(2):
            c = 2 * c2 + b
            drain(b)
            rows = pl.ds(obase + c * _BBLK, _BBLK)
            pltpu.sync_copy(ebufs[b], out_hbm.at[rows, pl.ds(0, SIZE_P)])
            pltpu.sync_copy(obufs[b], out_hbm.at[rows, pl.ds(SIZE_P, SIZE_P)])

            @pl.when(c + 2 < nq)
            def _():
                fire(c + 2, b)

        return carry

    lax.fori_loop(0, nq // 2, step, 0)


def _sc_gather(ctr4, table, phase):
    t0, tp = _PHASES[phase]
    mesh = plsc.VectorSubcoreMesh(core_axis_name="c", subcore_axis_name="s")
    k = pl.kernel(
        functools.partial(_sc_body, t0, tp),
        jax.ShapeDtypeStruct((_NW * 4 * tp * _BBLK, 2 * SIZE_P), jnp.float32),
        mesh=mesh,
        scratch_types=[
            pltpu.VMEM((tp, 8, _BBLK), jnp.float32),
            pltpu.VMEM((tp, 8, _BBLK), jnp.int32),
            pltpu.VMEM((_BBLK, SIZE_P), jnp.float32),
            pltpu.VMEM((_BBLK, SIZE_P), jnp.float32),
            pltpu.VMEM((_BBLK, SIZE_P), jnp.float32),
            pltpu.VMEM((_BBLK, SIZE_P), jnp.float32),
            pltpu.SemaphoreType.DMA,
            pltpu.SemaphoreType.DMA,
        ],
        compiler_params=pltpu.CompilerParams(use_tc_tiling_on_sc=False),
    )
    return k(ctr4, table)


def _tc_transpose(x128, phase, prev=None):
    t0, tp = _PHASES[phase]
    nq = 4 * tp

    def body(x_ref, o_ref):
        for q in range(nq):
            blk = x_ref[pl.ds(q * _BBLK, _BBLK), :].T
            o_ref[pl.ds(2 * q, 2), :, :] = blk.reshape(2, SIZE_P, _BBLK)

    def body_aliased(x_ref, prev_ref, o_ref):
        del prev_ref
        body(x_ref, o_ref)

    out_shape = jax.ShapeDtypeStruct((MAX_CLICKED, SIZE_P, BATCH), jnp.float32)
    x_spec = pl.BlockSpec((nq * _BBLK, 2 * SIZE_P), lambda i: (i, 0))
    o_spec = pl.BlockSpec((2 * nq, SIZE_P, _BBLK),
                          lambda i, p=8 * t0 // (2 * nq): (p, 0, i))
    if prev is None:
        return pl.pallas_call(
            body, grid=(_NW,), in_specs=[x_spec], out_specs=o_spec,
            out_shape=out_shape,
        )(x128)
    return pl.pallas_call(
        body_aliased, grid=(_NW,),
        in_specs=[x_spec, pl.BlockSpec(memory_space=pltpu.MemorySpace.HBM)],
        out_specs=o_spec, out_shape=out_shape,
        input_output_aliases={1: 0},
    )(x128, prev)


@jax.jit
def kernel(ctr, embedding_table):
    ctr4 = ctr.reshape(32, 128, 25, 8).transpose(2, 0, 3, 1)
    inters = [_sc_gather(ctr4, embedding_table, p) for p in range(len(_PHASES))]
    phys = _tc_transpose(inters[0], 0)
    for p in range(1, len(_PHASES)):
        phys = _tc_transpose(inters[p], p, prev=phys)
    return phys.transpose(2, 0, 1)
